# Initial kernel scaffold; baseline (speedup 1.0000x reference)
import jax
import jax.numpy as jnp
from jax.experimental import pallas as pl


def kernel(x, edge_types, means, stds, mul_w, bias_w):
    raise NotImplementedError("write your pallas kernel here")


# speedup vs baseline: 63.3084x; 63.3084x over previous
import functools

import jax
import jax.numpy as jnp
import numpy as np
from jax import lax
from jax.experimental import pallas as pl
from jax.experimental.pallas import tpu as pltpu
from jax.experimental.pallas import tpu_sc as plsc

_NW = 32
_LANES = 16


def _sc_gather_fma(et0, et1, xflat, mul_tab, bias_tab):
    P = xflat.shape[0]
    E = mul_tab.shape[0]
    CH = P // _NW
    mesh = plsc.VectorSubcoreMesh(core_axis_name="c", subcore_axis_name="s")

    @functools.partial(
        pl.kernel,
        out_type=jax.ShapeDtypeStruct((P,), jnp.float32),
        mesh=mesh,
        compiler_params=pltpu.CompilerParams(needs_layout_passes=False),
        scratch_types=[
            pltpu.VMEM((CH,), jnp.int32),
            pltpu.VMEM((CH,), jnp.int32),
            pltpu.VMEM((CH,), jnp.float32),
            pltpu.VMEM((CH,), jnp.float32),
            pltpu.VMEM((E,), jnp.float32),
            pltpu.VMEM((E,), jnp.float32),
        ],
    )
    def sc_kernel(et0_hbm, et1_hbm, x_hbm, mw_hbm, bw_hbm, out_hbm,
                  i0_v, i1_v, x_v, o_v, mw_v, bw_v):
        wid = lax.axis_index("s") * 2 + lax.axis_index("c")
        base = wid * CH
        pltpu.sync_copy(mw_hbm, mw_v)
        pltpu.sync_copy(bw_hbm, bw_v)
        pltpu.sync_copy(et0_hbm.at[pl.ds(base, CH)], i0_v)
        pltpu.sync_copy(et1_hbm.at[pl.ds(base, CH)], i1_v)
        pltpu.sync_copy(x_hbm.at[pl.ds(base, CH)], x_v)

        def body(i, carry):
            s = pl.ds(i * _LANES, _LANES)
            a = i0_v[s]
            b = i1_v[s]
            m = plsc.load_gather(mw_v, [a]) + plsc.load_gather(mw_v, [b])
            bias = plsc.load_gather(bw_v, [a]) + plsc.load_gather(bw_v, [b])
            o_v[s] = m * x_v[s] + bias
            return carry

        lax.fori_loop(0, CH // _LANES, body, 0)
        pltpu.sync_copy(o_v, out_hbm.at[pl.ds(base, CH)])

    return sc_kernel(et0, et1, xflat, mul_tab, bias_tab)


def _tc_expand(xe2d, means, stds):
    Q, NC = xe2d.shape
    K = means.shape[-1]
    RQ = 8

    def body(xe_ref, mean_ref, std_ref, out_ref):
        std = jnp.abs(std_ref[...]) + 0.01
        inv = (1.0 / std).reshape(1, 1, K)
        coef = (inv * np.float32(1.0 / np.sqrt(2.0 * np.pi))).reshape(1, 1, K)
        mean = mean_ref[...].reshape(1, 1, K)
        xe = xe_ref[...][..., None]
        t = (xe - mean) * inv
        out_ref[...] = jnp.exp(t * t * np.float32(-0.5)) * coef

    return pl.pallas_call(
        body,
        grid=(Q // RQ,),
        in_specs=[
            pl.BlockSpec((RQ, NC), lambda q: (q, 0)),
            pl.BlockSpec((1, K), lambda q: (0, 0)),
            pl.BlockSpec((1, K), lambda q: (0, 0)),
        ],
        out_specs=pl.BlockSpec((RQ, NC, K), lambda q: (q, 0, 0)),
        out_shape=jax.ShapeDtypeStruct((Q, NC, K), jnp.float32),
    )(xe2d, means, stds)


def kernel(x, edge_types, means, stds, mul_w, bias_w):
    B, N = x.shape[0], x.shape[1]
    K = means.shape[-1]
    et = edge_types.astype(jnp.int32)
    et0 = et[..., 0].reshape(-1)
    et1 = et[..., 1].reshape(-1)
    xe = _sc_gather_fma(et0, et1, x.reshape(-1),
                        mul_w.reshape(-1).astype(jnp.float32),
                        bias_w.reshape(-1).astype(jnp.float32))
    out = _tc_expand(xe.reshape(B * N, N),
                     means.astype(jnp.float32), stds.astype(jnp.float32))
    return out.reshape(B, N, N, K).astype(means.dtype)
